# baseline (speedup 1.0000x reference)
import functools

import jax
import jax.numpy as jnp
from jax import lax
from jax.experimental import pallas as pl
from jax.experimental.pallas import tpu as pltpu
from jax.experimental.pallas import tpu_sc as plsc

B = 16384
DIM = 64
NC = 2
NS = 16
NW = NC * NS
BPW = B // NW
CH = 128
NCH = BPW // CH


def _sc_gather(users, items, user_latent, item_latent):
    mesh = plsc.VectorSubcoreMesh(core_axis_name="c", subcore_axis_name="s")

    @functools.partial(
        pl.kernel,
        out_type=[
            jax.ShapeDtypeStruct((NW, NCH, CH, DIM), jnp.float32),
            jax.ShapeDtypeStruct((NW, NCH, CH, DIM), jnp.float32),
        ],
        mesh=mesh,
        scratch_types=[
            pltpu.VMEM((NCH, CH), jnp.int32),
            pltpu.VMEM((NCH, CH), jnp.int32),
            pltpu.VMEM((NCH, CH, DIM), jnp.float32),
            pltpu.VMEM((NCH, CH, DIM), jnp.float32),
            pltpu.SemaphoreType.DMA,
        ],
        compiler_params=pltpu.CompilerParams(use_tc_tiling_on_sc=False),
    )
    def k(users_hbm, items_hbm, ul_hbm, il_hbm, uout_hbm, iout_hbm,
          uidx_v, iidx_v, urows_v, irows_v, sem):
        wid = lax.axis_index("s") * NC + lax.axis_index("c")
        pltpu.sync_copy(users_hbm.at[wid], uidx_v)
        pltpu.sync_copy(items_hbm.at[wid], iidx_v)
        copies = []
        for c in range(NCH):
            copies.append(
                pltpu.async_copy(ul_hbm.at[uidx_v.at[c]], urows_v.at[c], sem))
            copies.append(
                pltpu.async_copy(il_hbm.at[iidx_v.at[c]], irows_v.at[c], sem))
        for cp in copies:
            cp.wait()
        pltpu.sync_copy(urows_v, uout_hbm.at[wid])
        pltpu.sync_copy(irows_v, iout_hbm.at[wid])

    u_rows, i_rows = k(
        users.reshape(NW, NCH, CH), items.reshape(NW, NCH, CH),
        user_latent, item_latent)
    return u_rows.reshape(B, DIM), i_rows.reshape(B, DIM)


BLK = 2048


def _tc_mlp(u_rows, i_rows, w1t, b1_row, w2_row):
    def body(u_ref, i_ref, w1t_ref, b1_ref, w2_ref, out_ref):
        z = jnp.concatenate([u_ref[...], i_ref[...]], axis=1)
        h = lax.dot_general(z, w1t_ref[...], (((1,), (0,)), ((), ())),
                            preferred_element_type=jnp.float32)
        h = jnp.maximum(h + b1_ref[...], 0.0)
        out_ref[...] = jnp.sum(h * w2_ref[...], axis=1, keepdims=True)

    out = pl.pallas_call(
        body,
        grid=(B // BLK,),
        in_specs=[
            pl.BlockSpec((BLK, DIM), lambda i: (i, 0)),
            pl.BlockSpec((BLK, DIM), lambda i: (i, 0)),
            pl.BlockSpec((2 * DIM, DIM), lambda i: (0, 0)),
            pl.BlockSpec((1, DIM), lambda i: (0, 0)),
            pl.BlockSpec((1, DIM), lambda i: (0, 0)),
        ],
        out_specs=pl.BlockSpec((BLK, 1), lambda i: (i, 0)),
        out_shape=jax.ShapeDtypeStruct((B, 1), jnp.float32),
    )(u_rows, i_rows, w1t, b1_row, w2_row)
    return out


def kernel(users, items, user_latent, item_latent, W1, b1, W2):
    u_rows, i_rows = _sc_gather(users, items, user_latent, item_latent)
    w1t = W1.T
    b1_row = b1.reshape(1, DIM)
    w2_row = W2.reshape(1, DIM)
    out = _tc_mlp(u_rows, i_rows, w1t, b1_row, w2_row)
    return out.reshape(B)
